# baseline (speedup 1.0000x reference)
import functools

import jax
import jax.numpy as jnp
from jax import lax
from jax.experimental import pallas as pl
from jax.experimental.pallas import tpu as pltpu
from jax.experimental.pallas import tpu_sc as plsc

N = 10000
E = 320000
D_IN = 128
D_HID = 16
D_OUT = 64

N_PAD = 10240
CHUNK = 128
CH_FULL = 78
TAIL = 16
EROWS = 2500

_mesh = plsc.VectorSubcoreMesh(core_axis_name="c", subcore_axis_name="s")
_sc_params = pltpu.CompilerParams(use_tc_tiling_on_sc=False)


def _stage_indices(ei_hbm, row, w, idx_v, tail_v, sem):
    pltpu.async_copy(ei_hbm.at[row, pl.ds(w * CH_FULL, CH_FULL)], idx_v, sem)
    pltpu.async_copy(
        ei_hbm.at[row, 32 * CH_FULL + w // 8, pl.ds((w % 8) * TAIL, TAIL)],
        tail_v, sem)


def _wait_indices(ei_hbm, row, w, idx_v, tail_v, sem):
    pltpu.make_async_copy(
        ei_hbm.at[row, pl.ds(w * CH_FULL, CH_FULL)], idx_v, sem).wait()
    pltpu.make_async_copy(
        ei_hbm.at[row, 32 * CH_FULL + w // 8, pl.ds((w % 8) * TAIL, TAIL)],
        tail_v, sem).wait()


@functools.partial(
    pl.kernel,
    out_type=jax.ShapeDtypeStruct((2, N_PAD), jnp.float32),
    mesh=_mesh,
    scratch_types=[
        pltpu.VMEM((CH_FULL, CHUNK), jnp.int32),
        pltpu.VMEM((TAIL,), jnp.int32),
        pltpu.VMEM((CHUNK,), jnp.float32),
        pltpu.VMEM((TAIL,), jnp.float32),
        pltpu.VMEM((640,), jnp.float32),
        pltpu.VMEM_SHARED((N_PAD,), jnp.float32),
        pltpu.SemaphoreType.DMA,
        pltpu.SemaphoreType.DMA,
    ],
    compiler_params=_sc_params,
)
def _count_kernel(ei_hbm, out_hbm, didx_v, dtail_v, ones_v, ones_t, zbuf_v,
                  acc_sh, isem, ssem):
    c = lax.axis_index("c")
    s = lax.axis_index("s")
    w = c * 16 + s
    _stage_indices(ei_hbm, 1, w, didx_v, dtail_v, isem)

    def fill_body(i, _):
        ones_v[pl.ds(i * 16, 16)] = jnp.ones((16,), jnp.float32)
        zbuf_v[pl.ds(i * 16, 16)] = jnp.zeros((16,), jnp.float32)
        zbuf_v[pl.ds((i + 8) * 16, 16)] = jnp.zeros((16,), jnp.float32)
        return 0
    lax.fori_loop(0, CHUNK // 16, fill_body, 0)
    ones_t[...] = jnp.ones((16,), jnp.float32)

    def zero_body(i, _):
        zbuf_v[pl.ds(256 + i * 16, 16)] = jnp.zeros((16,), jnp.float32)
        return 0
    lax.fori_loop(0, (640 - 256) // 16, zero_body, 0)

    _wait_indices(ei_hbm, 1, w, didx_v, dtail_v, isem)
    pltpu.sync_copy(zbuf_v, acc_sh.at[pl.ds(s * 640, 640)])
    plsc.subcore_barrier()

    def cnt_body(j, _):
        pltpu.make_async_copy(ones_v, acc_sh.at[didx_v.at[j]], ssem).wait()
        pltpu.async_copy(ones_v, acc_sh.at[didx_v.at[j + 4]], ssem, add=True)
        return 0

    for j0 in range(4):
        pltpu.async_copy(ones_v, acc_sh.at[didx_v.at[j0]], ssem, add=True)
    lax.fori_loop(0, CH_FULL - 4, cnt_body, 0)

    def drain_body(j, _):
        pltpu.make_async_copy(ones_v, acc_sh.at[didx_v.at[j]], ssem).wait()
        return 0
    lax.fori_loop(0, 4, drain_body, 0)
    pltpu.sync_copy(ones_t, acc_sh.at[dtail_v], add=True)
    plsc.subcore_barrier()

    pltpu.sync_copy(acc_sh.at[pl.ds(s * 640, 640)],
                    out_hbm.at[c].at[pl.ds(s * 640, 640)])


def _make_agg_kernel(d_feat):
    @functools.partial(
        pl.kernel,
        out_type=jax.ShapeDtypeStruct((2, N_PAD, d_feat), jnp.float32),
        mesh=_mesh,
        scratch_types=[
            pltpu.VMEM((CH_FULL, CHUNK), jnp.int32),
            pltpu.VMEM((CH_FULL, CHUNK), jnp.int32),
            pltpu.VMEM((TAIL,), jnp.int32),
            pltpu.VMEM((TAIL,), jnp.int32),
            pltpu.VMEM((4, CHUNK, d_feat), jnp.float32),
            pltpu.VMEM((TAIL, d_feat), jnp.float32),
            pltpu.VMEM((64, d_feat), jnp.float32),
            pltpu.VMEM_SHARED((N_PAD, d_feat), jnp.float32),
            pltpu.SemaphoreType.DMA,
            pltpu.SemaphoreType.DMA,
            pltpu.SemaphoreType.DMA,
            pltpu.SemaphoreType.DMA,
            pltpu.SemaphoreType.DMA,
        ],
        compiler_params=_sc_params,
    )
    def agg(ei_hbm, g_hbm, out_hbm, sidx_v, didx_v, stail_v, dtail_v, rows_v,
            trows_v, zbuf_v, acc_sh, isem, g0, g1, g2, g3):
        gsem = (g0, g1, g2, g3)
        c = lax.axis_index("c")
        s = lax.axis_index("s")
        w = c * 16 + s
        _stage_indices(ei_hbm, 0, w, sidx_v, stail_v, isem)
        _stage_indices(ei_hbm, 1, w, didx_v, dtail_v, isem)

        def zzero(i, _):
            j = i // (d_feat // 16)
            k = i % (d_feat // 16)
            zbuf_v[j, pl.ds(k * 16, 16)] = jnp.zeros((16,), jnp.float32)
            return 0
        lax.fori_loop(0, 64 * (d_feat // 16), zzero, 0)

        _wait_indices(ei_hbm, 0, w, sidx_v, stail_v, isem)
        _wait_indices(ei_hbm, 1, w, didx_v, dtail_v, isem)

        for t in range(10):
            pltpu.async_copy(zbuf_v, acc_sh.at[pl.ds(s * 640 + t * 64, 64)],
                             isem)
        for t in range(10):
            pltpu.make_async_copy(
                zbuf_v, acc_sh.at[pl.ds(s * 640 + t * 64, 64)], isem).wait()
        plsc.subcore_barrier()

        def issue(i, b):
            pltpu.async_copy(g_hbm.at[sidx_v.at[i]], rows_v.at[b], gsem[b])

        def drain(i, b):
            pltpu.make_async_copy(g_hbm.at[sidx_v.at[i]], rows_v.at[b],
                                  gsem[b]).wait()
            pltpu.sync_copy(rows_v.at[b], acc_sh.at[didx_v.at[i]], add=True)

        for b in range(4):
            issue(b, b)

        def outer_body(o, _):
            for b in range(4):
                i = o * 4 + b
                drain(i, b)
                issue(i + 4, b)
            return 0
        lax.fori_loop(0, CH_FULL // 4 - 1, outer_body, 0)

        base = (CH_FULL // 4 - 1) * 4
        drain(base + 0, 0)
        issue(base + 4, 0)
        drain(base + 1, 1)
        issue(base + 5, 1)
        drain(base + 2, 2)
        drain(base + 3, 3)
        drain(base + 4, 0)
        drain(base + 5, 1)

        pltpu.async_copy(g_hbm.at[stail_v], trows_v, isem)
        pltpu.make_async_copy(g_hbm.at[stail_v], trows_v, isem).wait()
        pltpu.sync_copy(trows_v, acc_sh.at[dtail_v], add=True)
        plsc.subcore_barrier()

        pltpu.sync_copy(acc_sh.at[pl.ds(s * 640, 640)],
                        out_hbm.at[c].at[pl.ds(s * 640, 640)])
    return agg


_agg16 = _make_agg_kernel(D_HID)
_agg64 = _make_agg_kernel(D_OUT)


def _dis_body(cnt_ref, out_ref):
    deg = jnp.sum(cnt_ref[...], axis=0, keepdims=True) + 1.0
    out_ref[...] = lax.rsqrt(deg)


_dis_kernel = pl.pallas_call(
    _dis_body,
    out_shape=jax.ShapeDtypeStruct((1, N_PAD), jnp.float32),
)


_BLK = 1000
_NBLK = N // _BLK


def _g1_body(x_ref, w1_ref, dis_ref, out_ref):
    h = jnp.dot(x_ref[...], w1_ref[...], preferred_element_type=jnp.float32)
    out_ref[...] = dis_ref[...] * h


_g1_kernel = pl.pallas_call(
    _g1_body,
    grid=(_NBLK,),
    in_specs=[
        pl.BlockSpec((_BLK, D_IN), lambda i: (i, 0)),
        pl.BlockSpec((D_IN, D_HID), lambda i: (0, 0)),
        pl.BlockSpec((_BLK, 1), lambda i: (i, 0)),
    ],
    out_specs=pl.BlockSpec((_BLK, D_HID), lambda i: (i, 0)),
    out_shape=jax.ShapeDtypeStruct((N, D_HID), jnp.float32),
)


def _mid_body(p_ref, g1_ref, dis_ref, b1_ref, w2_ref, out_ref):
    agg = p_ref[0] + p_ref[1]
    a1 = jnp.maximum(dis_ref[...] * (agg + g1_ref[...]) + b1_ref[...], 0.0)
    h2 = jnp.dot(a1, w2_ref[...], preferred_element_type=jnp.float32)
    out_ref[...] = dis_ref[...] * h2


_mid_kernel = pl.pallas_call(
    _mid_body,
    grid=(_NBLK,),
    in_specs=[
        pl.BlockSpec((2, _BLK, D_HID), lambda i: (0, i, 0)),
        pl.BlockSpec((_BLK, D_HID), lambda i: (i, 0)),
        pl.BlockSpec((_BLK, 1), lambda i: (i, 0)),
        pl.BlockSpec((1, D_HID), lambda i: (0, 0)),
        pl.BlockSpec((D_HID, D_OUT), lambda i: (0, 0)),
    ],
    out_specs=pl.BlockSpec((_BLK, D_OUT), lambda i: (i, 0)),
    out_shape=jax.ShapeDtypeStruct((N, D_OUT), jnp.float32),
)


def _fin_body(q_ref, g2_ref, dis_ref, b2_ref, out_ref):
    agg = q_ref[0] + q_ref[1]
    out_ref[...] = dis_ref[...] * (agg + g2_ref[...]) + b2_ref[...]


_fin_kernel = pl.pallas_call(
    _fin_body,
    grid=(_NBLK,),
    in_specs=[
        pl.BlockSpec((2, _BLK, D_OUT), lambda i: (0, i, 0)),
        pl.BlockSpec((_BLK, D_OUT), lambda i: (i, 0)),
        pl.BlockSpec((_BLK, 1), lambda i: (i, 0)),
        pl.BlockSpec((1, D_OUT), lambda i: (0, 0)),
    ],
    out_specs=pl.BlockSpec((_BLK, D_OUT), lambda i: (i, 0)),
    out_shape=jax.ShapeDtypeStruct((N, D_OUT), jnp.float32),
)


def kernel(x, edge_index, W1, b1, W2, b2):
    ei = edge_index.astype(jnp.int32).reshape(2, EROWS, CHUNK)

    cntp = _count_kernel(ei)
    dis_row = _dis_kernel(cntp)
    dis_col = dis_row.reshape(N_PAD, 1)[:N]

    g1 = _g1_kernel(x, W1, dis_col)
    p1 = _agg16(ei, g1)
    g2 = _mid_kernel(p1, g1, dis_col, b1.reshape(1, D_HID), W2)
    p2 = _agg64(ei, g2)
    out = _fin_kernel(p2, g2, dis_col, b2.reshape(1, D_OUT))
    return out


# speedup vs baseline: 1.0359x; 1.0359x over previous
import functools

import jax
import jax.numpy as jnp
from jax import lax
from jax.experimental import pallas as pl
from jax.experimental.pallas import tpu as pltpu
from jax.experimental.pallas import tpu_sc as plsc

N = 10000
E = 320000
D_IN = 128
D_HID = 16
D_OUT = 64

N_PAD = 10240
CHUNK = 128
CH_FULL = 78
TAIL = 16
EROWS = 2500

_mesh = plsc.VectorSubcoreMesh(core_axis_name="c", subcore_axis_name="s")
_sc_params = pltpu.CompilerParams(use_tc_tiling_on_sc=False)


def _stage_indices(ei_hbm, row, w, idx_v, tail_v, sem):
    pltpu.async_copy(ei_hbm.at[row, pl.ds(w * CH_FULL, CH_FULL)], idx_v, sem)
    pltpu.async_copy(
        ei_hbm.at[row, 32 * CH_FULL + w // 8, pl.ds((w % 8) * TAIL, TAIL)],
        tail_v, sem)


def _wait_indices(ei_hbm, row, w, idx_v, tail_v, sem):
    pltpu.make_async_copy(
        ei_hbm.at[row, pl.ds(w * CH_FULL, CH_FULL)], idx_v, sem).wait()
    pltpu.make_async_copy(
        ei_hbm.at[row, 32 * CH_FULL + w // 8, pl.ds((w % 8) * TAIL, TAIL)],
        tail_v, sem).wait()


@functools.partial(
    pl.kernel,
    out_type=jax.ShapeDtypeStruct((2, N_PAD), jnp.float32),
    mesh=_mesh,
    scratch_types=[
        pltpu.VMEM((CH_FULL, CHUNK), jnp.int32),
        pltpu.VMEM((TAIL,), jnp.int32),
        pltpu.VMEM((CHUNK,), jnp.float32),
        pltpu.VMEM((TAIL,), jnp.float32),
        pltpu.VMEM((640,), jnp.float32),
        pltpu.VMEM_SHARED((N_PAD,), jnp.float32),
        pltpu.SemaphoreType.DMA,
        pltpu.SemaphoreType.DMA,
    ],
    compiler_params=_sc_params,
)
def _count_kernel(ei_hbm, out_hbm, didx_v, dtail_v, ones_v, ones_t, zbuf_v,
                  acc_sh, isem, ssem):
    c = lax.axis_index("c")
    s = lax.axis_index("s")
    w = c * 16 + s
    _stage_indices(ei_hbm, 1, w, didx_v, dtail_v, isem)

    def fill_body(i, _):
        ones_v[pl.ds(i * 16, 16)] = jnp.ones((16,), jnp.float32)
        zbuf_v[pl.ds(i * 16, 16)] = jnp.zeros((16,), jnp.float32)
        zbuf_v[pl.ds((i + 8) * 16, 16)] = jnp.zeros((16,), jnp.float32)
        return 0
    lax.fori_loop(0, CHUNK // 16, fill_body, 0)
    ones_t[...] = jnp.ones((16,), jnp.float32)

    def zero_body(i, _):
        zbuf_v[pl.ds(256 + i * 16, 16)] = jnp.zeros((16,), jnp.float32)
        return 0
    lax.fori_loop(0, (640 - 256) // 16, zero_body, 0)

    _wait_indices(ei_hbm, 1, w, didx_v, dtail_v, isem)
    pltpu.sync_copy(zbuf_v, acc_sh.at[pl.ds(s * 640, 640)])
    plsc.subcore_barrier()

    def cnt_body(j, _):
        pltpu.make_async_copy(ones_v, acc_sh.at[didx_v.at[j]], ssem).wait()
        pltpu.async_copy(ones_v, acc_sh.at[didx_v.at[j + 4]], ssem, add=True)
        return 0

    for j0 in range(4):
        pltpu.async_copy(ones_v, acc_sh.at[didx_v.at[j0]], ssem, add=True)
    lax.fori_loop(0, CH_FULL - 4, cnt_body, 0)

    def drain_body(j, _):
        pltpu.make_async_copy(ones_v, acc_sh.at[didx_v.at[j]], ssem).wait()
        return 0
    lax.fori_loop(0, 4, drain_body, 0)
    pltpu.sync_copy(ones_t, acc_sh.at[dtail_v], add=True)
    plsc.subcore_barrier()

    pltpu.sync_copy(acc_sh.at[pl.ds(s * 640, 640)],
                    out_hbm.at[c].at[pl.ds(s * 640, 640)])


def _make_agg_kernel(d_feat):
    @functools.partial(
        pl.kernel,
        out_type=jax.ShapeDtypeStruct((2, N_PAD, d_feat), jnp.float32),
        mesh=_mesh,
        scratch_types=[
            pltpu.VMEM((CH_FULL, CHUNK), jnp.int32),
            pltpu.VMEM((CH_FULL, CHUNK), jnp.int32),
            pltpu.VMEM((TAIL,), jnp.int32),
            pltpu.VMEM((TAIL,), jnp.int32),
            pltpu.VMEM((4, CHUNK, d_feat), jnp.float32),
            pltpu.VMEM((TAIL, d_feat), jnp.float32),
            pltpu.VMEM((64, d_feat), jnp.float32),
            pltpu.VMEM_SHARED((N_PAD, d_feat), jnp.float32),
            pltpu.SemaphoreType.DMA,
            pltpu.SemaphoreType.DMA,
            pltpu.SemaphoreType.DMA,
            pltpu.SemaphoreType.DMA,
            pltpu.SemaphoreType.DMA,
        ],
        compiler_params=_sc_params,
    )
    def agg(ei_hbm, g_hbm, out_hbm, sidx_v, didx_v, stail_v, dtail_v, rows_v,
            trows_v, zbuf_v, acc_sh, isem, g0, g1, g2, g3):
        gsem = (g0, g1, g2, g3)
        c = lax.axis_index("c")
        s = lax.axis_index("s")
        w = c * 16 + s
        _stage_indices(ei_hbm, 0, w, sidx_v, stail_v, isem)
        _stage_indices(ei_hbm, 1, w, didx_v, dtail_v, isem)

        def zzero(i, _):
            j = i // (d_feat // 16)
            k = i % (d_feat // 16)
            zbuf_v[j, pl.ds(k * 16, 16)] = jnp.zeros((16,), jnp.float32)
            return 0
        lax.fori_loop(0, 64 * (d_feat // 16), zzero, 0)

        _wait_indices(ei_hbm, 0, w, sidx_v, stail_v, isem)
        _wait_indices(ei_hbm, 1, w, didx_v, dtail_v, isem)

        for t in range(10):
            pltpu.async_copy(zbuf_v, acc_sh.at[pl.ds(s * 640 + t * 64, 64)],
                             isem)
        for t in range(10):
            pltpu.make_async_copy(
                zbuf_v, acc_sh.at[pl.ds(s * 640 + t * 64, 64)], isem).wait()
        plsc.subcore_barrier()

        def issue(i, b):
            pltpu.async_copy(g_hbm.at[sidx_v.at[i]], rows_v.at[b], gsem[b])

        def drain(i, b):
            pltpu.make_async_copy(g_hbm.at[sidx_v.at[i]], rows_v.at[b],
                                  gsem[b]).wait()
            pltpu.sync_copy(rows_v.at[b], acc_sh.at[didx_v.at[i]], add=True)

        for b in range(4):
            issue(b, b)

        def outer_body(o, _):
            for b in range(4):
                i = o * 4 + b
                drain(i, b)
                issue(i + 4, b)
            return 0
        lax.fori_loop(0, CH_FULL // 4 - 1, outer_body, 0)

        base = (CH_FULL // 4 - 1) * 4
        drain(base + 0, 0)
        issue(base + 4, 0)
        drain(base + 1, 1)
        issue(base + 5, 1)
        drain(base + 2, 2)
        drain(base + 3, 3)
        drain(base + 4, 0)
        drain(base + 5, 1)

        pltpu.async_copy(g_hbm.at[stail_v], trows_v, isem)
        pltpu.make_async_copy(g_hbm.at[stail_v], trows_v, isem).wait()
        pltpu.sync_copy(trows_v, acc_sh.at[dtail_v], add=True)
        plsc.subcore_barrier()

        pltpu.sync_copy(acc_sh.at[pl.ds(s * 640, 640)],
                        out_hbm.at[c].at[pl.ds(s * 640, 640)])
    return agg


_agg16 = _make_agg_kernel(D_HID)
_agg64 = _make_agg_kernel(D_OUT)


def _dis_body(cnt_ref, out_ref):
    deg = jnp.sum(cnt_ref[...], axis=0, keepdims=True) + 1.0
    out_ref[...] = lax.rsqrt(deg)


_dis_kernel = pl.pallas_call(
    _dis_body,
    out_shape=jax.ShapeDtypeStruct((1, N_PAD), jnp.float32),
)


_BLK = 2000
_NBLK = N // _BLK


def _g1_body(x_ref, w1_ref, dis_ref, out_ref):
    h = jnp.dot(x_ref[...], w1_ref[...], preferred_element_type=jnp.float32)
    out_ref[...] = dis_ref[...] * h


_g1_kernel = pl.pallas_call(
    _g1_body,
    grid=(_NBLK,),
    in_specs=[
        pl.BlockSpec((_BLK, D_IN), lambda i: (i, 0)),
        pl.BlockSpec((D_IN, D_HID), lambda i: (0, 0)),
        pl.BlockSpec((_BLK, 1), lambda i: (i, 0)),
    ],
    out_specs=pl.BlockSpec((_BLK, D_HID), lambda i: (i, 0)),
    out_shape=jax.ShapeDtypeStruct((N, D_HID), jnp.float32),
)


def _mid_body(p_ref, g1_ref, dis_ref, b1_ref, w2_ref, out_ref):
    agg = p_ref[0] + p_ref[1]
    a1 = jnp.maximum(dis_ref[...] * (agg + g1_ref[...]) + b1_ref[...], 0.0)
    h2 = jnp.dot(a1, w2_ref[...], preferred_element_type=jnp.float32)
    out_ref[...] = dis_ref[...] * h2


_mid_kernel = pl.pallas_call(
    _mid_body,
    grid=(_NBLK,),
    in_specs=[
        pl.BlockSpec((2, _BLK, D_HID), lambda i: (0, i, 0)),
        pl.BlockSpec((_BLK, D_HID), lambda i: (i, 0)),
        pl.BlockSpec((_BLK, 1), lambda i: (i, 0)),
        pl.BlockSpec((1, D_HID), lambda i: (0, 0)),
        pl.BlockSpec((D_HID, D_OUT), lambda i: (0, 0)),
    ],
    out_specs=pl.BlockSpec((_BLK, D_OUT), lambda i: (i, 0)),
    out_shape=jax.ShapeDtypeStruct((N, D_OUT), jnp.float32),
)


def _fin_body(q_ref, g2_ref, dis_ref, b2_ref, out_ref):
    agg = q_ref[0] + q_ref[1]
    out_ref[...] = dis_ref[...] * (agg + g2_ref[...]) + b2_ref[...]


_fin_kernel = pl.pallas_call(
    _fin_body,
    grid=(_NBLK,),
    in_specs=[
        pl.BlockSpec((2, _BLK, D_OUT), lambda i: (0, i, 0)),
        pl.BlockSpec((_BLK, D_OUT), lambda i: (i, 0)),
        pl.BlockSpec((_BLK, 1), lambda i: (i, 0)),
        pl.BlockSpec((1, D_OUT), lambda i: (0, 0)),
    ],
    out_specs=pl.BlockSpec((_BLK, D_OUT), lambda i: (i, 0)),
    out_shape=jax.ShapeDtypeStruct((N, D_OUT), jnp.float32),
)


def kernel(x, edge_index, W1, b1, W2, b2):
    ei = edge_index.astype(jnp.int32).reshape(2, EROWS, CHUNK)

    cntp = _count_kernel(ei)
    dis_row = _dis_kernel(cntp)
    dis_col = dis_row.reshape(N_PAD, 1)[:N]

    g1 = _g1_kernel(x, W1, dis_col)
    p1 = _agg16(ei, g1)
    g2 = _mid_kernel(p1, g1, dis_col, b1.reshape(1, D_HID), W2)
    p2 = _agg64(ei, g2)
    out = _fin_kernel(p2, g2, dis_col, b2.reshape(1, D_OUT))
    return out


# speedup vs baseline: 1.0606x; 1.0238x over previous
import functools

import jax
import jax.numpy as jnp
from jax import lax
from jax.experimental import pallas as pl
from jax.experimental.pallas import tpu as pltpu
from jax.experimental.pallas import tpu_sc as plsc

N = 10000
E = 320000
D_IN = 128
D_HID = 16
D_OUT = 64

N_PAD = 10240
CHUNK = 128
CH_FULL = 78
TAIL = 16
EROWS = 2500

_mesh = plsc.VectorSubcoreMesh(core_axis_name="c", subcore_axis_name="s")
_sc_params = pltpu.CompilerParams(use_tc_tiling_on_sc=False)


def _stage_indices(ei_hbm, row, w, idx_v, tail_v, sem):
    pltpu.async_copy(ei_hbm.at[row, pl.ds(w * CH_FULL, CH_FULL)], idx_v, sem)
    pltpu.async_copy(
        ei_hbm.at[row, 32 * CH_FULL + w // 8, pl.ds((w % 8) * TAIL, TAIL)],
        tail_v, sem)


def _wait_indices(ei_hbm, row, w, idx_v, tail_v, sem):
    pltpu.make_async_copy(
        ei_hbm.at[row, pl.ds(w * CH_FULL, CH_FULL)], idx_v, sem).wait()
    pltpu.make_async_copy(
        ei_hbm.at[row, 32 * CH_FULL + w // 8, pl.ds((w % 8) * TAIL, TAIL)],
        tail_v, sem).wait()


@functools.partial(
    pl.kernel,
    out_type=jax.ShapeDtypeStruct((2, N_PAD), jnp.float32),
    mesh=_mesh,
    scratch_types=[
        pltpu.VMEM((CH_FULL, CHUNK), jnp.int32),
        pltpu.VMEM((TAIL,), jnp.int32),
        pltpu.VMEM((CHUNK,), jnp.float32),
        pltpu.VMEM((TAIL,), jnp.float32),
        pltpu.VMEM((640,), jnp.float32),
        pltpu.VMEM_SHARED((N_PAD,), jnp.float32),
        pltpu.SemaphoreType.DMA,
        pltpu.SemaphoreType.DMA,
    ],
    compiler_params=_sc_params,
)
def _count_kernel(ei_hbm, out_hbm, didx_v, dtail_v, ones_v, ones_t, zbuf_v,
                  acc_sh, isem, ssem):
    c = lax.axis_index("c")
    s = lax.axis_index("s")
    w = c * 16 + s
    _stage_indices(ei_hbm, 1, w, didx_v, dtail_v, isem)

    def fill_body(i, _):
        ones_v[pl.ds(i * 16, 16)] = jnp.ones((16,), jnp.float32)
        zbuf_v[pl.ds(i * 16, 16)] = jnp.zeros((16,), jnp.float32)
        zbuf_v[pl.ds((i + 8) * 16, 16)] = jnp.zeros((16,), jnp.float32)
        return 0
    lax.fori_loop(0, CHUNK // 16, fill_body, 0)
    ones_t[...] = jnp.ones((16,), jnp.float32)

    def zero_body(i, _):
        zbuf_v[pl.ds(256 + i * 16, 16)] = jnp.zeros((16,), jnp.float32)
        return 0
    lax.fori_loop(0, (640 - 256) // 16, zero_body, 0)

    _wait_indices(ei_hbm, 1, w, didx_v, dtail_v, isem)
    pltpu.sync_copy(zbuf_v, acc_sh.at[pl.ds(s * 640, 640)])
    plsc.subcore_barrier()

    def cnt_body(j, _):
        pltpu.make_async_copy(ones_v, acc_sh.at[didx_v.at[j]], ssem).wait()
        pltpu.async_copy(ones_v, acc_sh.at[didx_v.at[j + 4]], ssem, add=True)
        return 0

    for j0 in range(4):
        pltpu.async_copy(ones_v, acc_sh.at[didx_v.at[j0]], ssem, add=True)
    lax.fori_loop(0, CH_FULL - 4, cnt_body, 0)

    def drain_body(j, _):
        pltpu.make_async_copy(ones_v, acc_sh.at[didx_v.at[j]], ssem).wait()
        return 0
    lax.fori_loop(0, 4, drain_body, 0)
    pltpu.sync_copy(ones_t, acc_sh.at[dtail_v], add=True)
    plsc.subcore_barrier()

    pltpu.sync_copy(acc_sh.at[pl.ds(s * 640, 640)],
                    out_hbm.at[c].at[pl.ds(s * 640, 640)])


def _make_agg_kernel(d_feat):
    @functools.partial(
        pl.kernel,
        out_type=jax.ShapeDtypeStruct((2, N_PAD, d_feat), jnp.float32),
        mesh=_mesh,
        scratch_types=[
            pltpu.VMEM((CH_FULL, CHUNK), jnp.int32),
            pltpu.VMEM((CH_FULL, CHUNK), jnp.int32),
            pltpu.VMEM((TAIL,), jnp.int32),
            pltpu.VMEM((TAIL,), jnp.int32),
            pltpu.VMEM((4, CHUNK, d_feat), jnp.float32),
            pltpu.VMEM((TAIL, d_feat), jnp.float32),
            pltpu.VMEM((64, d_feat), jnp.float32),
            pltpu.VMEM_SHARED((N_PAD, d_feat), jnp.float32),
            pltpu.SemaphoreType.DMA,
            pltpu.SemaphoreType.DMA,
            pltpu.SemaphoreType.DMA,
            pltpu.SemaphoreType.DMA,
            pltpu.SemaphoreType.DMA,
        ],
        compiler_params=_sc_params,
    )
    def agg(ei_hbm, g_hbm, out_hbm, sidx_v, didx_v, stail_v, dtail_v, rows_v,
            trows_v, zbuf_v, acc_sh, isem, g0, g1, g2, g3):
        gsem = (g0, g1, g2, g3)
        c = lax.axis_index("c")
        s = lax.axis_index("s")
        w = c * 16 + s
        _stage_indices(ei_hbm, 0, w, sidx_v, stail_v, isem)
        _stage_indices(ei_hbm, 1, w, didx_v, dtail_v, isem)

        def zzero(i, _):
            j = i // (d_feat // 16)
            k = i % (d_feat // 16)
            zbuf_v[j, pl.ds(k * 16, 16)] = jnp.zeros((16,), jnp.float32)
            return 0
        lax.fori_loop(0, 64 * (d_feat // 16), zzero, 0)

        _wait_indices(ei_hbm, 0, w, sidx_v, stail_v, isem)
        _wait_indices(ei_hbm, 1, w, didx_v, dtail_v, isem)

        for t in range(10):
            pltpu.async_copy(zbuf_v, acc_sh.at[pl.ds(s * 640 + t * 64, 64)],
                             isem)
        for t in range(10):
            pltpu.make_async_copy(
                zbuf_v, acc_sh.at[pl.ds(s * 640 + t * 64, 64)], isem).wait()
        plsc.subcore_barrier()

        def issue(i, b):
            pltpu.async_copy(g_hbm.at[sidx_v.at[i]], rows_v.at[b], gsem[b])

        def drain(i, b):
            pltpu.make_async_copy(g_hbm.at[sidx_v.at[i]], rows_v.at[b],
                                  gsem[b]).wait()
            pltpu.sync_copy(rows_v.at[b], acc_sh.at[didx_v.at[i]], add=True)

        for b in range(4):
            issue(b, b)

        def outer_body(o, _):
            for b in range(4):
                i = o * 4 + b
                drain(i, b)
                issue(i + 4, b)
            return 0
        lax.fori_loop(0, CH_FULL // 4 - 1, outer_body, 0)

        base = (CH_FULL // 4 - 1) * 4
        drain(base + 0, 0)
        issue(base + 4, 0)
        drain(base + 1, 1)
        issue(base + 5, 1)
        drain(base + 2, 2)
        drain(base + 3, 3)
        drain(base + 4, 0)
        drain(base + 5, 1)

        pltpu.async_copy(g_hbm.at[stail_v], trows_v, isem)
        pltpu.make_async_copy(g_hbm.at[stail_v], trows_v, isem).wait()
        pltpu.sync_copy(trows_v, acc_sh.at[dtail_v], add=True)
        plsc.subcore_barrier()

        pltpu.sync_copy(acc_sh.at[pl.ds(s * 640, 640)],
                        out_hbm.at[c].at[pl.ds(s * 640, 640)])
    return agg


_agg16 = _make_agg_kernel(D_HID)
_agg64 = _make_agg_kernel(D_OUT)


def _h1_body(x_ref, w1_ref, out_ref):
    out_ref[...] = jnp.dot(x_ref[...], w1_ref[...],
                           preferred_element_type=jnp.float32)


_h1_kernel = pl.pallas_call(
    _h1_body,
    out_shape=jax.ShapeDtypeStruct((N, D_HID), jnp.float32),
)


def _scale_body(cnt_ref, h1_ref, g1_ref, dis_ref):
    deg = jnp.sum(cnt_ref[...], axis=0, keepdims=True) + 1.0
    dis_col = jnp.transpose(lax.rsqrt(deg), (1, 0))
    dis_ref[...] = dis_col
    g1_ref[...] = dis_col[:N] * h1_ref[...]


_scale_kernel = pl.pallas_call(
    _scale_body,
    out_shape=(jax.ShapeDtypeStruct((N, D_HID), jnp.float32),
               jax.ShapeDtypeStruct((N_PAD, 1), jnp.float32)),
)


def _mid_body(p_ref, g1_ref, dis_ref, b1_ref, w2_ref, out_ref):
    agg = p_ref[0, :N, :] + p_ref[1, :N, :]
    a1 = jnp.maximum(dis_ref[...] * (agg + g1_ref[...]) + b1_ref[...], 0.0)
    h2 = jnp.dot(a1, w2_ref[...], preferred_element_type=jnp.float32)
    out_ref[...] = dis_ref[...] * h2


_mid_kernel = pl.pallas_call(
    _mid_body,
    out_shape=jax.ShapeDtypeStruct((N, D_OUT), jnp.float32),
)


def _fin_body(q_ref, g2_ref, dis_ref, b2_ref, out_ref):
    agg = q_ref[0, :N, :] + q_ref[1, :N, :]
    out_ref[...] = dis_ref[...] * (agg + g2_ref[...]) + b2_ref[...]


_fin_kernel = pl.pallas_call(
    _fin_body,
    out_shape=jax.ShapeDtypeStruct((N, D_OUT), jnp.float32),
)


def kernel(x, edge_index, W1, b1, W2, b2):
    ei = edge_index.astype(jnp.int32).reshape(2, EROWS, CHUNK)

    cntp = _count_kernel(ei)
    h1 = _h1_kernel(x, W1)
    g1, dis_col_p = _scale_kernel(cntp, h1)
    dis_col = dis_col_p[:N]

    p1 = _agg16(ei, g1)
    g2 = _mid_kernel(p1, g1, dis_col, b1.reshape(1, D_HID), W2)
    p2 = _agg64(ei, g2)
    out = _fin_kernel(p2, g2, dis_col, b2.reshape(1, D_OUT))
    return out
